# baseline (speedup 1.0000x reference)
import functools

import jax
import jax.numpy as jnp
from jax import lax
from jax.experimental import pallas as pl
from jax.experimental.pallas import tpu as pltpu
from jax.experimental.pallas import tpu_sc as plsc

N = 10000
D = 128
E = 320000
NC, NS, L = 2, 16, 16
NW = NC * NS
K = 128
NCH = 80
NCHT = 2560
PC = 8
CPW0 = 88
CPW1 = 72
EPAD = NW * NCH * K
NP = 10240
RPT = NP // NS

_mesh = plsc.VectorSubcoreMesh(core_axis_name="c", subcore_axis_name="s")


@jax.jit
def _sc_degree(dst_r, zeros_t, ones_t):
    @functools.partial(
        pl.kernel,
        out_type=jax.ShapeDtypeStruct((NC, NP, D), jnp.float32),
        mesh=_mesh,
        scratch_types=[
            pltpu.VMEM((NCH, K), jnp.int32),
            pltpu.VMEM((K, D), jnp.float32),
            pltpu.VMEM_SHARED((NP, D), jnp.float32),
        ],
    )
    def k(dst_hbm, z_hbm, ones_hbm, out_hbm, didx, ones_v, acc):
        cid = lax.axis_index("c")
        sid = lax.axis_index("s")
        wid = cid * NS + sid
        pltpu.sync_copy(z_hbm.at[pl.ds(sid * RPT, RPT)],
                        acc.at[pl.ds(sid * RPT, RPT)])
        pltpu.sync_copy(ones_hbm, ones_v)
        pltpu.sync_copy(dst_hbm.at[wid], didx)
        plsc.subcore_barrier()

        @pl.loop(0, NCH)
        def _(j):
            pltpu.sync_copy(ones_v, acc.at[didx.at[j]], add=True)

        plsc.subcore_barrier()
        pltpu.sync_copy(acc.at[pl.ds(sid * RPT, RPT)],
                        out_hbm.at[cid, pl.ds(sid * RPT, RPT)])

    return k(dst_r, zeros_t, ones_t)


@jax.jit
def _sc_aggregate(g, src_r, dst_r):
    @functools.partial(
        pl.kernel,
        out_type=jax.ShapeDtypeStruct((NC, NP, D), jnp.float32),
        mesh=_mesh,
        scratch_types=[
            pltpu.VMEM((PC, K), jnp.int32),
            pltpu.VMEM((PC, K), jnp.int32),
            pltpu.VMEM((K, D), jnp.float32),
            pltpu.VMEM((K, D), jnp.float32),
            pltpu.VMEM_SHARED((NP, D), jnp.float32),
            pltpu.SemaphoreType.DMA,
            pltpu.SemaphoreType.DMA,
            pltpu.SemaphoreType.DMA,
            pltpu.SemaphoreType.DMA,
        ],
    )
    def k(g_hbm, src_hbm, dst_hbm, out_hbm, sidx, didx, rows0, rows1,
          acc, gs0, gs1, ss0, ss1):
        rows = (rows0, rows1)
        gs = (gs0, gs1)
        ss = (ss0, ss1)
        cid = lax.axis_index("c")
        sid = lax.axis_index("s")
        pltpu.sync_copy(g_hbm.at[pl.ds(sid * RPT, RPT)],
                        acc.at[pl.ds(sid * RPT, RPT)])
        plsc.subcore_barrier()

        def piece_loop(base_chunk, npieces):
            @pl.loop(0, npieces)
            def _(p):
                c0 = pl.multiple_of(base_chunk + p * PC, 8)
                pltpu.sync_copy(src_hbm.at[pl.ds(c0, PC)], sidx)
                pltpu.sync_copy(dst_hbm.at[pl.ds(c0, PC)], didx)
                pltpu.async_copy(g_hbm.at[sidx.at[0]], rows[0], gs[0])

                @pl.loop(0, PC, step=2)
                def _(j0):
                    for b in range(2):
                        j = j0 + b
                        bp = 1 - b
                        pltpu.make_async_copy(g_hbm.at[sidx.at[j]], rows[b],
                                              gs[b]).wait()
                        pltpu.async_copy(rows[b], acc.at[didx.at[j]], ss[b],
                                         add=True)

                        @pl.when(j > 0)
                        def _():
                            pltpu.make_async_copy(rows[bp],
                                                  acc.at[didx.at[j - 1]],
                                                  ss[bp]).wait()

                        @pl.when(j + 1 < PC)
                        def _():
                            pltpu.async_copy(g_hbm.at[sidx.at[j + 1]],
                                             rows[bp], gs[bp])

                pltpu.make_async_copy(rows[(PC - 1) % 2],
                                      acc.at[didx.at[PC - 1]],
                                      ss[(PC - 1) % 2]).wait()

        @pl.when(cid == 0)
        def _():
            piece_loop(sid * CPW0, CPW0 // PC)

        @pl.when(cid == 1)
        def _():
            piece_loop(16 * CPW0 + sid * CPW1, CPW1 // PC)

        plsc.subcore_barrier()
        pltpu.sync_copy(acc.at[pl.ds(sid * RPT, RPT)],
                        out_hbm.at[cid, pl.ds(sid * RPT, RPT)])

    return k(g, src_r, dst_r)


BR = 2000
GRID = N // BR


def _dinv_of(dp0, dp1):
    deg = dp0[0, :, 0:1] + dp1[0, :, 0:1] + 1.0
    return lax.rsqrt(deg)


def _tc_first_body(x_ref, w_ref, dp0_ref, dp1_ref, g_ref):
    dinv = _dinv_of(dp0_ref[...], dp1_ref[...])
    g_ref[...] = jnp.dot(x_ref[...], w_ref[...],
                         preferred_element_type=jnp.float32) * dinv


def _tc_mid_body(pa_ref, pb_ref, g_ref, dp0_ref, dp1_ref, b_ref, w_ref, o_ref):
    dinv = _dinv_of(dp0_ref[...], dp1_ref[...])
    a = (pa_ref[0] + pb_ref[0] - g_ref[...]) * dinv + b_ref[...]
    a = jnp.maximum(a, 0.0)
    o_ref[...] = jnp.dot(a, w_ref[...],
                         preferred_element_type=jnp.float32) * dinv


def _tc_final_body(pa_ref, pb_ref, g_ref, dp0_ref, dp1_ref, b_ref, o_ref):
    dinv = _dinv_of(dp0_ref[...], dp1_ref[...])
    o_ref[...] = (pa_ref[0] + pb_ref[0] - g_ref[...]) * dinv + b_ref[...]


_spec_rows = pl.BlockSpec((BR, D), lambda i: (i, 0))
_spec_w = pl.BlockSpec((D, D), lambda i: (0, 0))
_spec_b = pl.BlockSpec((1, D), lambda i: (0, 0))
_spec_p0 = pl.BlockSpec((1, BR, D), lambda i: (0, i, 0))
_spec_p1 = pl.BlockSpec((1, BR, D), lambda i: (1, i, 0))
_spec_dp0 = pl.BlockSpec((1, BR, D), lambda i: (0, i, 0))
_spec_dp1 = pl.BlockSpec((1, BR, D), lambda i: (1, i, 0))
_out_rows = jax.ShapeDtypeStruct((N, D), jnp.float32)
_out_g = jax.ShapeDtypeStruct((NP, D), jnp.float32)

_tc_first = pl.pallas_call(
    _tc_first_body, grid=(GRID,),
    in_specs=[_spec_rows, _spec_w, _spec_dp0, _spec_dp1],
    out_specs=_spec_rows, out_shape=_out_g)

_tc_mid = pl.pallas_call(
    _tc_mid_body, grid=(GRID,),
    in_specs=[_spec_p0, _spec_p1, _spec_rows, _spec_dp0, _spec_dp1,
              _spec_b, _spec_w],
    out_specs=_spec_rows, out_shape=_out_g)

_tc_final = pl.pallas_call(
    _tc_final_body, grid=(GRID,),
    in_specs=[_spec_p0, _spec_p1, _spec_rows, _spec_dp0, _spec_dp1, _spec_b],
    out_specs=_spec_rows, out_shape=_out_rows)


def kernel(x, edge_index, state, W1, b1, W2, b2, W3, b3):
    src = edge_index[0].astype(jnp.int32)
    dst = edge_index[1].astype(jnp.int32)
    pad = EPAD - E
    src_r = jnp.concatenate(
        [src, jnp.zeros((pad,), jnp.int32)]).reshape(NCHT, K)
    dst_r2 = jnp.concatenate(
        [dst, jnp.full((pad,), N, jnp.int32)]).reshape(NCHT, K)
    dst_r = dst_r2.reshape(NW, NCH, K)

    zeros_t = jnp.zeros((NP, D), jnp.float32)
    ones_t = jnp.ones((K, D), jnp.float32)
    dp = _sc_degree(dst_r, zeros_t, ones_t)

    b1r, b2r, b3r = (b.reshape(1, D) for b in (b1, b2, b3))

    g1 = _tc_first(x, W1, dp, dp)
    p1 = _sc_aggregate(g1, src_r, dst_r2)
    g2 = _tc_mid(p1, p1, g1, dp, dp, b1r, W2)
    p2 = _sc_aggregate(g2, src_r, dst_r2)
    g3 = _tc_mid(p2, p2, g2, dp, dp, b2r, W3)
    p3 = _sc_aggregate(g3, src_r, dst_r2)
    out = _tc_final(p3, p3, g3, dp, dp, b3r)
    return out


# speedup vs baseline: 1.0388x; 1.0388x over previous
import functools

import jax
import jax.numpy as jnp
from jax import lax
from jax.experimental import pallas as pl
from jax.experimental.pallas import tpu as pltpu
from jax.experimental.pallas import tpu_sc as plsc

N = 10000
D = 128
E = 320000
NC, NS, L = 2, 16, 16
NW = NC * NS
K = 128
NCH = 80
NCHT = 2560
PC = 32
CPW0 = 96
CPW1 = 64
EPAD = NW * NCH * K
NP = 10240
RPT = NP // NS

_mesh = plsc.VectorSubcoreMesh(core_axis_name="c", subcore_axis_name="s")


@jax.jit
def _sc_degree(dst_r, zeros_t, ones_t):
    @functools.partial(
        pl.kernel,
        out_type=jax.ShapeDtypeStruct((NC, NP, D), jnp.float32),
        mesh=_mesh,
        scratch_types=[
            pltpu.VMEM((NCH, K), jnp.int32),
            pltpu.VMEM((K, D), jnp.float32),
            pltpu.VMEM_SHARED((NP, D), jnp.float32),
            pltpu.SemaphoreType.DMA,
        ],
    )
    def k(dst_hbm, z_hbm, ones_hbm, out_hbm, didx, ones_v, acc, sem):
        cid = lax.axis_index("c")
        sid = lax.axis_index("s")
        wid = cid * NS + sid
        pltpu.sync_copy(z_hbm.at[pl.ds(sid * RPT, RPT)],
                        acc.at[pl.ds(sid * RPT, RPT)])
        pltpu.sync_copy(ones_hbm, ones_v)
        pltpu.sync_copy(dst_hbm.at[wid], didx)
        plsc.subcore_barrier()

        @pl.loop(0, NCH, step=8)
        def _(j0):
            for b in range(8):
                pltpu.async_copy(ones_v, acc.at[didx.at[j0 + b]], sem,
                                 add=True)
            for b in range(8):
                pltpu.make_async_copy(ones_v, acc.at[didx.at[j0 + b]],
                                      sem).wait()

        plsc.subcore_barrier()
        pltpu.sync_copy(acc.at[pl.ds(sid * RPT, RPT)],
                        out_hbm.at[cid, pl.ds(sid * RPT, RPT)])

    return k(dst_r, zeros_t, ones_t)


@jax.jit
def _sc_aggregate(g, src_r, dst_r):
    @functools.partial(
        pl.kernel,
        out_type=jax.ShapeDtypeStruct((NC, NP, D), jnp.float32),
        mesh=_mesh,
        scratch_types=[
            pltpu.VMEM((PC, K), jnp.int32),
            pltpu.VMEM((PC, K), jnp.int32),
            pltpu.VMEM((K, D), jnp.float32),
            pltpu.VMEM((K, D), jnp.float32),
            pltpu.VMEM_SHARED((NP, D), jnp.float32),
            pltpu.SemaphoreType.DMA,
            pltpu.SemaphoreType.DMA,
            pltpu.SemaphoreType.DMA,
            pltpu.SemaphoreType.DMA,
        ],
    )
    def k(g_hbm, src_hbm, dst_hbm, out_hbm, sidx, didx, rows0, rows1,
          acc, gs0, gs1, ss0, ss1):
        rows = (rows0, rows1)
        gs = (gs0, gs1)
        ss = (ss0, ss1)
        cid = lax.axis_index("c")
        sid = lax.axis_index("s")
        pltpu.sync_copy(g_hbm.at[pl.ds(sid * RPT, RPT)],
                        acc.at[pl.ds(sid * RPT, RPT)])
        plsc.subcore_barrier()

        def piece_loop(base_chunk, npieces):
            @pl.loop(0, npieces)
            def _(p):
                c0 = pl.multiple_of(base_chunk + p * PC, 8)
                pltpu.sync_copy(src_hbm.at[pl.ds(c0, PC)], sidx)
                pltpu.sync_copy(dst_hbm.at[pl.ds(c0, PC)], didx)
                pltpu.async_copy(g_hbm.at[sidx.at[0]], rows[0], gs[0])

                @pl.loop(0, PC, step=2)
                def _(j0):
                    for b in range(2):
                        j = j0 + b
                        bp = 1 - b
                        pltpu.make_async_copy(g_hbm.at[sidx.at[j]], rows[b],
                                              gs[b]).wait()
                        pltpu.async_copy(rows[b], acc.at[didx.at[j]], ss[b],
                                         add=True)

                        @pl.when(j > 0)
                        def _():
                            pltpu.make_async_copy(rows[bp],
                                                  acc.at[didx.at[j - 1]],
                                                  ss[bp]).wait()

                        @pl.when(j + 1 < PC)
                        def _():
                            pltpu.async_copy(g_hbm.at[sidx.at[j + 1]],
                                             rows[bp], gs[bp])

                pltpu.make_async_copy(rows[(PC - 1) % 2],
                                      acc.at[didx.at[PC - 1]],
                                      ss[(PC - 1) % 2]).wait()

        @pl.when(cid == 0)
        def _():
            piece_loop(sid * CPW0, CPW0 // PC)

        @pl.when(cid == 1)
        def _():
            piece_loop(16 * CPW0 + sid * CPW1, CPW1 // PC)

        plsc.subcore_barrier()
        pltpu.sync_copy(acc.at[pl.ds(sid * RPT, RPT)],
                        out_hbm.at[cid, pl.ds(sid * RPT, RPT)])

    return k(g, src_r, dst_r)


BR = 2000
GRID = N // BR


def _dinv_of(dp0, dp1):
    deg = dp0[0, :, 0:1] + dp1[0, :, 0:1] + 1.0
    return lax.rsqrt(deg)


def _tc_first_body(x_ref, w_ref, dp0_ref, dp1_ref, g_ref):
    dinv = _dinv_of(dp0_ref[...], dp1_ref[...])
    g_ref[...] = jnp.dot(x_ref[...], w_ref[...],
                         preferred_element_type=jnp.float32) * dinv


def _tc_mid_body(pa_ref, pb_ref, g_ref, dp0_ref, dp1_ref, b_ref, w_ref, o_ref):
    dinv = _dinv_of(dp0_ref[...], dp1_ref[...])
    a = (pa_ref[0] + pb_ref[0] - g_ref[...]) * dinv + b_ref[...]
    a = jnp.maximum(a, 0.0)
    o_ref[...] = jnp.dot(a, w_ref[...],
                         preferred_element_type=jnp.float32) * dinv


def _tc_final_body(pa_ref, pb_ref, g_ref, dp0_ref, dp1_ref, b_ref, o_ref):
    dinv = _dinv_of(dp0_ref[...], dp1_ref[...])
    o_ref[...] = (pa_ref[0] + pb_ref[0] - g_ref[...]) * dinv + b_ref[...]


_spec_rows = pl.BlockSpec((BR, D), lambda i: (i, 0))
_spec_w = pl.BlockSpec((D, D), lambda i: (0, 0))
_spec_b = pl.BlockSpec((1, D), lambda i: (0, 0))
_spec_p0 = pl.BlockSpec((1, BR, D), lambda i: (0, i, 0))
_spec_p1 = pl.BlockSpec((1, BR, D), lambda i: (1, i, 0))
_spec_dp0 = pl.BlockSpec((1, BR, D), lambda i: (0, i, 0))
_spec_dp1 = pl.BlockSpec((1, BR, D), lambda i: (1, i, 0))
_out_rows = jax.ShapeDtypeStruct((N, D), jnp.float32)
_out_g = jax.ShapeDtypeStruct((NP, D), jnp.float32)

_tc_first = pl.pallas_call(
    _tc_first_body, grid=(GRID,),
    in_specs=[_spec_rows, _spec_w, _spec_dp0, _spec_dp1],
    out_specs=_spec_rows, out_shape=_out_g)

_tc_mid = pl.pallas_call(
    _tc_mid_body, grid=(GRID,),
    in_specs=[_spec_p0, _spec_p1, _spec_rows, _spec_dp0, _spec_dp1,
              _spec_b, _spec_w],
    out_specs=_spec_rows, out_shape=_out_g)

_tc_final = pl.pallas_call(
    _tc_final_body, grid=(GRID,),
    in_specs=[_spec_p0, _spec_p1, _spec_rows, _spec_dp0, _spec_dp1, _spec_b],
    out_specs=_spec_rows, out_shape=_out_rows)


def kernel(x, edge_index, state, W1, b1, W2, b2, W3, b3):
    src = edge_index[0].astype(jnp.int32)
    dst = edge_index[1].astype(jnp.int32)
    pad = EPAD - E
    src_r = jnp.concatenate(
        [src, jnp.zeros((pad,), jnp.int32)]).reshape(NCHT, K)
    dst_r2 = jnp.concatenate(
        [dst, jnp.full((pad,), N, jnp.int32)]).reshape(NCHT, K)
    dst_r = dst_r2.reshape(NW, NCH, K)

    zeros_t = jnp.zeros((NP, D), jnp.float32)
    ones_t = jnp.ones((K, D), jnp.float32)
    dp = _sc_degree(dst_r, zeros_t, ones_t)

    b1r, b2r, b3r = (b.reshape(1, D) for b in (b1, b2, b3))

    g1 = _tc_first(x, W1, dp, dp)
    p1 = _sc_aggregate(g1, src_r, dst_r2)
    g2 = _tc_mid(p1, p1, g1, dp, dp, b1r, W2)
    p2 = _sc_aggregate(g2, src_r, dst_r2)
    g3 = _tc_mid(p2, p2, g2, dp, dp, b2r, W3)
    p3 = _sc_aggregate(g3, src_r, dst_r2)
    out = _tc_final(p3, p3, g3, dp, dp, b3r)
    return out


# speedup vs baseline: 1.0407x; 1.0018x over previous
import functools

import jax
import jax.numpy as jnp
from jax import lax
from jax.experimental import pallas as pl
from jax.experimental.pallas import tpu as pltpu
from jax.experimental.pallas import tpu_sc as plsc

N = 10000
D = 128
E = 320000
NC, NS, L = 2, 16, 16
NW = NC * NS
K = 128
NCH = 80
NCHT = 2560
PC = 32
CPW0 = 96
CPW1 = 64
EPAD = NW * NCH * K
NP = 10240
RPT = NP // NS

_mesh = plsc.VectorSubcoreMesh(core_axis_name="c", subcore_axis_name="s")


@jax.jit
def _sc_degree(dst_r, zeros_t, ones_t):
    @functools.partial(
        pl.kernel,
        out_type=jax.ShapeDtypeStruct((NC, NP, D), jnp.float32),
        mesh=_mesh,
        scratch_types=[
            pltpu.VMEM((NCH, K), jnp.int32),
            pltpu.VMEM((K, D), jnp.float32),
            pltpu.VMEM_SHARED((NP, D), jnp.float32),
            pltpu.SemaphoreType.DMA,
        ],
    )
    def k(dst_hbm, z_hbm, ones_hbm, out_hbm, didx, ones_v, acc, sem):
        cid = lax.axis_index("c")
        sid = lax.axis_index("s")
        wid = cid * NS + sid
        pltpu.sync_copy(z_hbm.at[pl.ds(sid * RPT, RPT)],
                        acc.at[pl.ds(sid * RPT, RPT)])
        pltpu.sync_copy(ones_hbm, ones_v)
        pltpu.sync_copy(dst_hbm.at[wid], didx)
        plsc.subcore_barrier()

        @pl.loop(0, NCH, step=8)
        def _(j0):
            for b in range(8):
                pltpu.async_copy(ones_v, acc.at[didx.at[j0 + b]], sem,
                                 add=True)
            for b in range(8):
                pltpu.make_async_copy(ones_v, acc.at[didx.at[j0 + b]],
                                      sem).wait()

        plsc.subcore_barrier()
        pltpu.sync_copy(acc.at[pl.ds(sid * RPT, RPT)],
                        out_hbm.at[cid, pl.ds(sid * RPT, RPT)])

    return k(dst_r, zeros_t, ones_t)


@jax.jit
def _sc_aggregate(g, src_r, dst_r):
    @functools.partial(
        pl.kernel,
        out_type=jax.ShapeDtypeStruct((NC, NP, D), jnp.float32),
        mesh=_mesh,
        scratch_types=[
            pltpu.VMEM((PC, K), jnp.int32),
            pltpu.VMEM((PC, K), jnp.int32),
            pltpu.VMEM((K, D), jnp.float32),
            pltpu.VMEM((K, D), jnp.float32),
            pltpu.VMEM_SHARED((NP, D), jnp.float32),
            pltpu.SemaphoreType.DMA,
            pltpu.SemaphoreType.DMA,
            pltpu.SemaphoreType.DMA,
            pltpu.SemaphoreType.DMA,
        ],
    )
    def k(g_hbm, src_hbm, dst_hbm, out_hbm, sidx, didx, rows0, rows1,
          acc, gs0, gs1, ss0, ss1):
        rows = (rows0, rows1)
        gs = (gs0, gs1)
        ss = (ss0, ss1)
        cid = lax.axis_index("c")
        sid = lax.axis_index("s")
        pltpu.sync_copy(g_hbm.at[pl.ds(sid * RPT, RPT)],
                        acc.at[pl.ds(sid * RPT, RPT)])
        plsc.subcore_barrier()

        def piece_loop(base_chunk, npieces):
            @pl.loop(0, npieces)
            def _(p):
                c0 = pl.multiple_of(base_chunk + p * PC, 8)
                pltpu.sync_copy(src_hbm.at[pl.ds(c0, PC)], sidx)
                pltpu.sync_copy(dst_hbm.at[pl.ds(c0, PC)], didx)
                pltpu.async_copy(g_hbm.at[sidx.at[0]], rows[0], gs[0])

                @pl.loop(0, PC, step=2)
                def _(j0):
                    for b in range(2):
                        j = j0 + b
                        bp = 1 - b
                        pltpu.make_async_copy(g_hbm.at[sidx.at[j]], rows[b],
                                              gs[b]).wait()
                        pltpu.async_copy(rows[b], acc.at[didx.at[j]], ss[b],
                                         add=True)

                        @pl.when(j > 0)
                        def _():
                            pltpu.make_async_copy(rows[bp],
                                                  acc.at[didx.at[j - 1]],
                                                  ss[bp]).wait()

                        @pl.when(j + 1 < PC)
                        def _():
                            pltpu.async_copy(g_hbm.at[sidx.at[j + 1]],
                                             rows[bp], gs[bp])

                pltpu.make_async_copy(rows[(PC - 1) % 2],
                                      acc.at[didx.at[PC - 1]],
                                      ss[(PC - 1) % 2]).wait()

        @pl.when(cid == 0)
        def _():
            piece_loop(sid * CPW0, CPW0 // PC)

        @pl.when(cid == 1)
        def _():
            piece_loop(16 * CPW0 + sid * CPW1, CPW1 // PC)

        plsc.subcore_barrier()
        pltpu.sync_copy(acc.at[pl.ds(sid * RPT, RPT)],
                        out_hbm.at[cid, pl.ds(sid * RPT, RPT)])

    return k(g, src_r, dst_r)


BR = 5000
GRID = N // BR


def _dinv_of(dp0, dp1):
    deg = dp0[0, :, 0:1] + dp1[0, :, 0:1] + 1.0
    return lax.rsqrt(deg)


def _tc_first_body(x_ref, w_ref, dp0_ref, dp1_ref, g_ref):
    dinv = _dinv_of(dp0_ref[...], dp1_ref[...])
    g_ref[...] = jnp.dot(x_ref[...], w_ref[...],
                         preferred_element_type=jnp.float32) * dinv


def _tc_mid_body(pa_ref, pb_ref, g_ref, dp0_ref, dp1_ref, b_ref, w_ref, o_ref):
    dinv = _dinv_of(dp0_ref[...], dp1_ref[...])
    a = (pa_ref[0] + pb_ref[0] - g_ref[...]) * dinv + b_ref[...]
    a = jnp.maximum(a, 0.0)
    o_ref[...] = jnp.dot(a, w_ref[...],
                         preferred_element_type=jnp.float32) * dinv


def _tc_final_body(pa_ref, pb_ref, g_ref, dp0_ref, dp1_ref, b_ref, o_ref):
    dinv = _dinv_of(dp0_ref[...], dp1_ref[...])
    o_ref[...] = (pa_ref[0] + pb_ref[0] - g_ref[...]) * dinv + b_ref[...]


_spec_rows = pl.BlockSpec((BR, D), lambda i: (i, 0))
_spec_w = pl.BlockSpec((D, D), lambda i: (0, 0))
_spec_b = pl.BlockSpec((1, D), lambda i: (0, 0))
_spec_p0 = pl.BlockSpec((1, BR, D), lambda i: (0, i, 0))
_spec_p1 = pl.BlockSpec((1, BR, D), lambda i: (1, i, 0))
_spec_dp0 = pl.BlockSpec((1, BR, D), lambda i: (0, i, 0))
_spec_dp1 = pl.BlockSpec((1, BR, D), lambda i: (1, i, 0))
_out_rows = jax.ShapeDtypeStruct((N, D), jnp.float32)
_out_g = jax.ShapeDtypeStruct((NP, D), jnp.float32)

_tc_first = pl.pallas_call(
    _tc_first_body, grid=(GRID,),
    in_specs=[_spec_rows, _spec_w, _spec_dp0, _spec_dp1],
    out_specs=_spec_rows, out_shape=_out_g)

_tc_mid = pl.pallas_call(
    _tc_mid_body, grid=(GRID,),
    in_specs=[_spec_p0, _spec_p1, _spec_rows, _spec_dp0, _spec_dp1,
              _spec_b, _spec_w],
    out_specs=_spec_rows, out_shape=_out_g)

_tc_final = pl.pallas_call(
    _tc_final_body, grid=(GRID,),
    in_specs=[_spec_p0, _spec_p1, _spec_rows, _spec_dp0, _spec_dp1, _spec_b],
    out_specs=_spec_rows, out_shape=_out_rows)


def kernel(x, edge_index, state, W1, b1, W2, b2, W3, b3):
    src = edge_index[0].astype(jnp.int32)
    dst = edge_index[1].astype(jnp.int32)
    pad = EPAD - E
    src_r = jnp.concatenate(
        [src, jnp.zeros((pad,), jnp.int32)]).reshape(NCHT, K)
    dst_r2 = jnp.concatenate(
        [dst, jnp.full((pad,), N, jnp.int32)]).reshape(NCHT, K)
    dst_r = dst_r2.reshape(NW, NCH, K)

    zeros_t = jnp.zeros((NP, D), jnp.float32)
    ones_t = jnp.ones((K, D), jnp.float32)
    dp = _sc_degree(dst_r, zeros_t, ones_t)

    b1r, b2r, b3r = (b.reshape(1, D) for b in (b1, b2, b3))

    g1 = _tc_first(x, W1, dp, dp)
    p1 = _sc_aggregate(g1, src_r, dst_r2)
    g2 = _tc_mid(p1, p1, g1, dp, dp, b1r, W2)
    p2 = _sc_aggregate(g2, src_r, dst_r2)
    g3 = _tc_mid(p2, p2, g2, dp, dp, b2r, W3)
    p3 = _sc_aggregate(g3, src_r, dst_r2)
    out = _tc_final(p3, p3, g3, dp, dp, b3r)
    return out
